# baseline (speedup 1.0000x reference)
import functools

import jax
import jax.numpy as jnp
import numpy as np
from jax.experimental import pallas as pl
from jax.experimental.pallas import tpu as pltpu

_NUM_CLASSES = 20
_GRID = 13
_NUM_ANCHORS = 5
_NSITE = _GRID * _GRID * _NUM_ANCHORS
_NOBJ = 20
_TB = 64
_ANCHORS = np.array(
    [[1.3221, 1.73145], [3.19275, 4.00944], [5.05587, 8.09892],
     [9.47112, 4.84053], [11.2364, 10.0071]], dtype=np.float32)


def _site_consts():
    a = np.arange(_NSITE, dtype=np.int32)
    anc = a // (_GRID * _GRID)
    cell = a % (_GRID * _GRID)
    ci = cell // _GRID
    cj = cell % _GRID
    rows = np.stack([
        a.astype(np.float32),
        ci.astype(np.float32),
        cj.astype(np.float32),
        _ANCHORS[anc, 0],
        _ANCHORS[anc, 1],
        np.zeros(_NSITE, np.float32),
    ], axis=0)
    return rows


def _loss_kernel(pred_ref, box_ref, lab_ref, site_ref, out_ref):
    b = pl.program_id(0)

    x0 = box_ref[0]
    y0 = box_ref[1]
    x1 = box_ref[2]
    y1 = box_ref[3]
    cgx0 = x0 * 13.0
    cgy0 = y0 * 13.0
    cgx1 = x1 * 13.0
    cgy1 = y1 * 13.0
    bcx = ((x0 + x1) / 2.0) * 13.0
    bcy = ((y0 + y1) / 2.0) * 13.0
    bw = (x1 - x0) * 13.0
    bh = (y1 - y0) * 13.0
    cxf = jnp.floor(bcx)
    cyf = jnp.floor(bcy)
    fx = bcx - cxf
    fy = bcy - cyf
    cxi = bcx.astype(jnp.int32)
    cyi = bcy.astype(jnp.int32)

    acx = cxf + 0.5
    acy = cyf + 0.5
    area_b = (cgx1 - cgx0) * (cgy1 - cgy0)
    best_iou = jnp.full_like(bcx, -1.0)
    best_j = jnp.zeros_like(cxi)
    for a in range(_NUM_ANCHORS):
        aw = float(_ANCHORS[a, 0])
        ah = float(_ANCHORS[a, 1])
        ax0 = acx - aw / 2.0
        ay0 = acy - ah / 2.0
        ax1 = acx + aw / 2.0
        ay1 = acy + ah / 2.0
        ltx = jnp.maximum(ax0, cgx0)
        lty = jnp.maximum(ay0, cgy0)
        rbx = jnp.minimum(ax1, cgx1)
        rby = jnp.minimum(ay1, cgy1)
        iw = jnp.maximum(rbx - ltx, 0.0)
        ih = jnp.maximum(rby - lty, 0.0)
        inter = iw * ih
        area_a = (ax1 - ax0) * (ay1 - ay0)
        iou_a = inter / (area_a + area_b - inter + 1e-10)
        better = iou_a > best_iou
        best_j = jnp.where(better, a, best_j)
        best_iou = jnp.where(better, iou_a, best_iou)
    aw_sel = jnp.zeros_like(bcx)
    ah_sel = jnp.zeros_like(bcx)
    for a in range(_NUM_ANCHORS):
        aw_sel = jnp.where(best_j == a, float(_ANCHORS[a, 0]), aw_sel)
        ah_sel = jnp.where(best_j == a, float(_ANCHORS[a, 1]), ah_sel)
    twv = bw / aw_sel
    thv = bh / ah_sel
    sid = best_j * (_GRID * _GRID) + cxi * _GRID + cyi

    flat_i = site_ref[0:1, :].astype(jnp.int32)
    labs = lab_ref[...]

    rm = jnp.zeros((_TB, _NSITE), jnp.float32)
    tx = jnp.zeros((_TB, _NSITE), jnp.float32)
    ty = jnp.zeros((_TB, _NSITE), jnp.float32)
    tw = jnp.zeros((_TB, _NSITE), jnp.float32)
    th = jnp.zeros((_TB, _NSITE), jnp.float32)
    cbits = jnp.zeros((_TB, _NSITE), jnp.int32)
    for n in range(_NOBJ):
        m = sid[:, n:n + 1] == flat_i
        rm = jnp.where(m, 1.0, rm)
        tx = jnp.where(m, fx[:, n:n + 1], tx)
        ty = jnp.where(m, fy[:, n:n + 1], ty)
        tw = jnp.where(m, twv[:, n:n + 1], tw)
        th = jnp.where(m, thv[:, n:n + 1], th)
        bit = jnp.left_shift(1, labs[:, n:n + 1])
        cbits = jnp.where(m, cbits | bit, cbits)

    px = jax.nn.sigmoid(pred_ref[0])
    py = jax.nn.sigmoid(pred_ref[1])
    pw = jnp.exp(pred_ref[2])
    ph = jnp.exp(pred_ref[3])
    pc = jax.nn.sigmoid(pred_ref[4])

    cell_x = site_ref[1:2, :]
    cell_y = site_ref[2:3, :]
    site_aw = site_ref[3:4, :]
    site_ah = site_ref[4:5, :]
    cpx = cell_x + px
    cpy = cell_y + py
    cpw = site_aw * pw
    cph = site_ah * ph
    px0 = cpx - cpw / 2.0
    py0 = cpy - cph / 2.0
    px1 = cpx + cpw / 2.0
    py1 = cpy + cph / 2.0
    parea = (px1 - px0) * (py1 - py0)
    gc = jnp.zeros((_TB, _NSITE), jnp.float32)
    for n in range(_NOBJ):
        gx0 = cgx0[:, n:n + 1]
        gy0 = cgy0[:, n:n + 1]
        gx1 = cgx1[:, n:n + 1]
        gy1 = cgy1[:, n:n + 1]
        iltx = jnp.maximum(px0, gx0)
        ilty = jnp.maximum(py0, gy0)
        irbx = jnp.minimum(px1, gx1)
        irby = jnp.minimum(py1, gy1)
        iiw = jnp.maximum(irbx - iltx, 0.0)
        iih = jnp.maximum(irby - ilty, 0.0)
        ii = iiw * iih
        ib = (gx1 - gx0) * (gy1 - gy0)
        gc = jnp.maximum(gc, ii / (parea + ib - ii + 1e-10))

    xy_l = jnp.sum(rm * ((tx - px) ** 2 + (ty - py) ** 2))
    wh_l = jnp.sum(rm * ((jnp.sqrt(tw) - jnp.sqrt(pw)) ** 2
                         + (jnp.sqrt(th) - jnp.sqrt(ph)) ** 2))
    d2 = (gc - pc) ** 2
    conf_l = jnp.sum(rm * d2)
    noconf_l = jnp.sum((1.0 - rm) * d2)

    mx = pred_ref[5]
    for c in range(1, _NUM_CLASSES):
        mx = jnp.maximum(mx, pred_ref[5 + c])
    es = []
    den = jnp.zeros((_TB, _NSITE), jnp.float32)
    for c in range(_NUM_CLASSES):
        e = jnp.exp(pred_ref[5 + c] - mx)
        es.append(e)
        den = den + e
    cls_sq = jnp.zeros((_TB, _NSITE), jnp.float32)
    for c in range(_NUM_CLASSES):
        tcls_c = (jnp.right_shift(cbits, c) & 1).astype(jnp.float32)
        cls_sq = cls_sq + (tcls_c - es[c] / den) ** 2
    rc = jnp.maximum(
        jnp.maximum(rm[:, 0:169], rm[:, 169:338]),
        jnp.maximum(rm[:, 338:507],
                    jnp.maximum(rm[:, 507:676], rm[:, 676:845])))
    rc845 = jnp.concatenate([rc, rc, rc, rc, rc], axis=1)
    cls_l = jnp.sum(rc845 * cls_sq)

    l1 = (5.0 * xy_l).reshape(1, 1)
    l2 = (5.0 * wh_l).reshape(1, 1)
    l3 = conf_l.reshape(1, 1)
    l4 = (0.5 * noconf_l).reshape(1, 1)
    l5 = cls_l.reshape(1, 1)
    tot = l1 + l2 + l3 + l4 + l5
    vec = jnp.concatenate([l1, l2, l3, l4, l5, tot, tot * 0.0, tot * 0.0],
                          axis=1)

    @pl.when(b == 0)
    def _():
        out_ref[...] = vec

    @pl.when(b != 0)
    def _():
        out_ref[...] = out_ref[...] + vec


@jax.jit
def kernel(pred_targets, gt_boxes, gt_labels):
    B = pred_targets.shape[0]
    pred = pred_targets.reshape(B, _GRID * _GRID, _NUM_ANCHORS, 25)
    pred = pred.transpose(3, 0, 2, 1).reshape(25, B, _NSITE)
    boxes = gt_boxes.astype(jnp.float32).transpose(2, 0, 1)
    labels = gt_labels.astype(jnp.int32)
    site = jnp.asarray(_site_consts())

    out = pl.pallas_call(
        _loss_kernel,
        grid=(B // _TB,),
        in_specs=[
            pl.BlockSpec((25, _TB, _NSITE), lambda b: (0, b, 0)),
            pl.BlockSpec((4, _TB, _NOBJ), lambda b: (0, b, 0)),
            pl.BlockSpec((_TB, _NOBJ), lambda b: (b, 0)),
            pl.BlockSpec((6, _NSITE), lambda b: (0, 0)),
        ],
        out_specs=pl.BlockSpec((1, 8), lambda b: (0, 0)),
        out_shape=jax.ShapeDtypeStruct((1, 8), jnp.float32),
    )(pred, boxes, labels, site)

    return (out[0, 5], out[0, 0], out[0, 1], out[0, 2], out[0, 3], out[0, 4])


# speedup vs baseline: 1.0127x; 1.0127x over previous
import functools

import jax
import jax.numpy as jnp
import numpy as np
from jax.experimental import pallas as pl
from jax.experimental.pallas import tpu as pltpu

_NUM_CLASSES = 20
_GRID = 13
_NUM_ANCHORS = 5
_NSITE = _GRID * _GRID * _NUM_ANCHORS
_NOBJ = 20
_TB = 32
_ANCHORS = np.array(
    [[1.3221, 1.73145], [3.19275, 4.00944], [5.05587, 8.09892],
     [9.47112, 4.84053], [11.2364, 10.0071]], dtype=np.float32)


def _site_consts():
    a = np.arange(_NSITE, dtype=np.int32)
    anc = a // (_GRID * _GRID)
    cell = a % (_GRID * _GRID)
    ci = cell // _GRID
    cj = cell % _GRID
    rows = np.stack([
        a.astype(np.float32),
        ci.astype(np.float32),
        cj.astype(np.float32),
        _ANCHORS[anc, 0],
        _ANCHORS[anc, 1],
        np.zeros(_NSITE, np.float32),
    ], axis=0)
    return rows


def _loss_kernel(pred_ref, box_ref, lab_ref, site_ref, out_ref):
    b = pl.program_id(0)

    x0 = box_ref[0]
    y0 = box_ref[1]
    x1 = box_ref[2]
    y1 = box_ref[3]
    cgx0 = x0 * 13.0
    cgy0 = y0 * 13.0
    cgx1 = x1 * 13.0
    cgy1 = y1 * 13.0
    bcx = ((x0 + x1) / 2.0) * 13.0
    bcy = ((y0 + y1) / 2.0) * 13.0
    bw = (x1 - x0) * 13.0
    bh = (y1 - y0) * 13.0
    cxf = jnp.floor(bcx)
    cyf = jnp.floor(bcy)
    fx = bcx - cxf
    fy = bcy - cyf
    cxi = bcx.astype(jnp.int32)
    cyi = bcy.astype(jnp.int32)

    acx = cxf + 0.5
    acy = cyf + 0.5
    area_b = (cgx1 - cgx0) * (cgy1 - cgy0)
    best_iou = jnp.full_like(bcx, -1.0)
    best_j = jnp.zeros_like(cxi)
    for a in range(_NUM_ANCHORS):
        aw = float(_ANCHORS[a, 0])
        ah = float(_ANCHORS[a, 1])
        ax0 = acx - aw / 2.0
        ay0 = acy - ah / 2.0
        ax1 = acx + aw / 2.0
        ay1 = acy + ah / 2.0
        ltx = jnp.maximum(ax0, cgx0)
        lty = jnp.maximum(ay0, cgy0)
        rbx = jnp.minimum(ax1, cgx1)
        rby = jnp.minimum(ay1, cgy1)
        iw = jnp.maximum(rbx - ltx, 0.0)
        ih = jnp.maximum(rby - lty, 0.0)
        inter = iw * ih
        area_a = (ax1 - ax0) * (ay1 - ay0)
        iou_a = inter / (area_a + area_b - inter + 1e-10)
        better = iou_a > best_iou
        best_j = jnp.where(better, a, best_j)
        best_iou = jnp.where(better, iou_a, best_iou)
    aw_sel = jnp.zeros_like(bcx)
    ah_sel = jnp.zeros_like(bcx)
    for a in range(_NUM_ANCHORS):
        aw_sel = jnp.where(best_j == a, float(_ANCHORS[a, 0]), aw_sel)
        ah_sel = jnp.where(best_j == a, float(_ANCHORS[a, 1]), ah_sel)
    twv = bw / aw_sel
    thv = bh / ah_sel
    sid = best_j * (_GRID * _GRID) + cxi * _GRID + cyi

    flat_i = site_ref[0:1, :].astype(jnp.int32)
    labs = lab_ref[...]

    rm = jnp.zeros((_TB, _NSITE), jnp.float32)
    tx = jnp.zeros((_TB, _NSITE), jnp.float32)
    ty = jnp.zeros((_TB, _NSITE), jnp.float32)
    tw = jnp.zeros((_TB, _NSITE), jnp.float32)
    th = jnp.zeros((_TB, _NSITE), jnp.float32)
    cbits = jnp.zeros((_TB, _NSITE), jnp.int32)
    for n in range(_NOBJ):
        m = sid[:, n:n + 1] == flat_i
        rm = jnp.where(m, 1.0, rm)
        tx = jnp.where(m, fx[:, n:n + 1], tx)
        ty = jnp.where(m, fy[:, n:n + 1], ty)
        tw = jnp.where(m, twv[:, n:n + 1], tw)
        th = jnp.where(m, thv[:, n:n + 1], th)
        bit = jnp.left_shift(1, labs[:, n:n + 1])
        cbits = jnp.where(m, cbits | bit, cbits)

    px = jax.nn.sigmoid(pred_ref[0])
    py = jax.nn.sigmoid(pred_ref[1])
    pw = jnp.exp(pred_ref[2])
    ph = jnp.exp(pred_ref[3])
    pc = jax.nn.sigmoid(pred_ref[4])

    cell_x = site_ref[1:2, :]
    cell_y = site_ref[2:3, :]
    site_aw = site_ref[3:4, :]
    site_ah = site_ref[4:5, :]
    cpx = cell_x + px
    cpy = cell_y + py
    cpw = site_aw * pw
    cph = site_ah * ph
    px0 = cpx - cpw / 2.0
    py0 = cpy - cph / 2.0
    px1 = cpx + cpw / 2.0
    py1 = cpy + cph / 2.0
    parea = (px1 - px0) * (py1 - py0)
    gc = jnp.zeros((_TB, _NSITE), jnp.float32)
    for n in range(_NOBJ):
        gx0 = cgx0[:, n:n + 1]
        gy0 = cgy0[:, n:n + 1]
        gx1 = cgx1[:, n:n + 1]
        gy1 = cgy1[:, n:n + 1]
        iltx = jnp.maximum(px0, gx0)
        ilty = jnp.maximum(py0, gy0)
        irbx = jnp.minimum(px1, gx1)
        irby = jnp.minimum(py1, gy1)
        iiw = jnp.maximum(irbx - iltx, 0.0)
        iih = jnp.maximum(irby - ilty, 0.0)
        ii = iiw * iih
        ib = (gx1 - gx0) * (gy1 - gy0)
        gc = jnp.maximum(gc, ii / (parea + ib - ii + 1e-10))

    xy_l = jnp.sum(rm * ((tx - px) ** 2 + (ty - py) ** 2))
    wh_l = jnp.sum(rm * ((jnp.sqrt(tw) - jnp.sqrt(pw)) ** 2
                         + (jnp.sqrt(th) - jnp.sqrt(ph)) ** 2))
    d2 = (gc - pc) ** 2
    conf_l = jnp.sum(rm * d2)
    noconf_l = jnp.sum((1.0 - rm) * d2)

    mx = pred_ref[5]
    for c in range(1, _NUM_CLASSES):
        mx = jnp.maximum(mx, pred_ref[5 + c])
    es = []
    den = jnp.zeros((_TB, _NSITE), jnp.float32)
    for c in range(_NUM_CLASSES):
        e = jnp.exp(pred_ref[5 + c] - mx)
        es.append(e)
        den = den + e
    cls_sq = jnp.zeros((_TB, _NSITE), jnp.float32)
    for c in range(_NUM_CLASSES):
        tcls_c = (jnp.right_shift(cbits, c) & 1).astype(jnp.float32)
        cls_sq = cls_sq + (tcls_c - es[c] / den) ** 2
    rc = jnp.maximum(
        jnp.maximum(rm[:, 0:169], rm[:, 169:338]),
        jnp.maximum(rm[:, 338:507],
                    jnp.maximum(rm[:, 507:676], rm[:, 676:845])))
    rc845 = jnp.concatenate([rc, rc, rc, rc, rc], axis=1)
    cls_l = jnp.sum(rc845 * cls_sq)

    l1 = (5.0 * xy_l).reshape(1, 1)
    l2 = (5.0 * wh_l).reshape(1, 1)
    l3 = conf_l.reshape(1, 1)
    l4 = (0.5 * noconf_l).reshape(1, 1)
    l5 = cls_l.reshape(1, 1)
    tot = l1 + l2 + l3 + l4 + l5
    vec = jnp.concatenate([l1, l2, l3, l4, l5, tot, tot * 0.0, tot * 0.0],
                          axis=1)

    @pl.when(b == 0)
    def _():
        out_ref[...] = vec

    @pl.when(b != 0)
    def _():
        out_ref[...] = out_ref[...] + vec


@jax.jit
def kernel(pred_targets, gt_boxes, gt_labels):
    B = pred_targets.shape[0]
    pred = pred_targets.reshape(B, _GRID * _GRID, _NUM_ANCHORS, 25)
    pred = pred.transpose(3, 0, 2, 1).reshape(25, B, _NSITE)
    boxes = gt_boxes.astype(jnp.float32).transpose(2, 0, 1)
    labels = gt_labels.astype(jnp.int32)
    site = jnp.asarray(_site_consts())

    out = pl.pallas_call(
        _loss_kernel,
        grid=(B // _TB,),
        in_specs=[
            pl.BlockSpec((25, _TB, _NSITE), lambda b: (0, b, 0)),
            pl.BlockSpec((4, _TB, _NOBJ), lambda b: (0, b, 0)),
            pl.BlockSpec((_TB, _NOBJ), lambda b: (b, 0)),
            pl.BlockSpec((6, _NSITE), lambda b: (0, 0)),
        ],
        out_specs=pl.BlockSpec((1, 8), lambda b: (0, 0)),
        out_shape=jax.ShapeDtypeStruct((1, 8), jnp.float32),
    )(pred, boxes, labels, site)

    return (out[0, 5], out[0, 0], out[0, 1], out[0, 2], out[0, 3], out[0, 4])


# speedup vs baseline: 1.8034x; 1.7808x over previous
import functools

import jax
import jax.numpy as jnp
import numpy as np
from jax.experimental import pallas as pl
from jax.experimental.pallas import tpu as pltpu

_NUM_CLASSES = 20
_GRID = 13
_NUM_ANCHORS = 5
_NSITE = _GRID * _GRID * _NUM_ANCHORS
_NOBJ = 20
_TB = 32
_ANCHORS = np.array(
    [[1.3221, 1.73145], [3.19275, 4.00944], [5.05587, 8.09892],
     [9.47112, 4.84053], [11.2364, 10.0071]], dtype=np.float32)


def _site_consts():
    a = np.arange(_NSITE, dtype=np.int32)
    anc = a // (_GRID * _GRID)
    cell = a % (_GRID * _GRID)
    ci = cell // _GRID
    cj = cell % _GRID
    rows = np.stack([
        a.astype(np.float32),
        ci.astype(np.float32),
        cj.astype(np.float32),
        _ANCHORS[anc, 0],
        _ANCHORS[anc, 1],
        np.zeros(_NSITE, np.float32),
    ], axis=0)
    return rows


def _loss_kernel(pred_ref, box_ref, lab_ref, site_ref, out_ref):
    b = pl.program_id(0)

    pt = jnp.transpose(pred_ref[...], (0, 2, 1))
    def chan(c):
        return jnp.concatenate([pt[:, a * 25 + c, :] for a in range(5)],
                               axis=1)

    x0 = box_ref[0]
    y0 = box_ref[1]
    x1 = box_ref[2]
    y1 = box_ref[3]
    cgx0 = x0 * 13.0
    cgy0 = y0 * 13.0
    cgx1 = x1 * 13.0
    cgy1 = y1 * 13.0
    bcx = ((x0 + x1) / 2.0) * 13.0
    bcy = ((y0 + y1) / 2.0) * 13.0
    bw = (x1 - x0) * 13.0
    bh = (y1 - y0) * 13.0
    cxf = jnp.floor(bcx)
    cyf = jnp.floor(bcy)
    fx = bcx - cxf
    fy = bcy - cyf
    cxi = bcx.astype(jnp.int32)
    cyi = bcy.astype(jnp.int32)

    acx = cxf + 0.5
    acy = cyf + 0.5
    area_b = (cgx1 - cgx0) * (cgy1 - cgy0)
    best_iou = jnp.full_like(bcx, -1.0)
    best_j = jnp.zeros_like(cxi)
    for a in range(_NUM_ANCHORS):
        aw = float(_ANCHORS[a, 0])
        ah = float(_ANCHORS[a, 1])
        ax0 = acx - aw / 2.0
        ay0 = acy - ah / 2.0
        ax1 = acx + aw / 2.0
        ay1 = acy + ah / 2.0
        ltx = jnp.maximum(ax0, cgx0)
        lty = jnp.maximum(ay0, cgy0)
        rbx = jnp.minimum(ax1, cgx1)
        rby = jnp.minimum(ay1, cgy1)
        iw = jnp.maximum(rbx - ltx, 0.0)
        ih = jnp.maximum(rby - lty, 0.0)
        inter = iw * ih
        area_a = (ax1 - ax0) * (ay1 - ay0)
        iou_a = inter / (area_a + area_b - inter + 1e-10)
        better = iou_a > best_iou
        best_j = jnp.where(better, a, best_j)
        best_iou = jnp.where(better, iou_a, best_iou)
    aw_sel = jnp.zeros_like(bcx)
    ah_sel = jnp.zeros_like(bcx)
    for a in range(_NUM_ANCHORS):
        aw_sel = jnp.where(best_j == a, float(_ANCHORS[a, 0]), aw_sel)
        ah_sel = jnp.where(best_j == a, float(_ANCHORS[a, 1]), ah_sel)
    twv = bw / aw_sel
    thv = bh / ah_sel
    sid = best_j * (_GRID * _GRID) + cxi * _GRID + cyi

    flat_i = site_ref[0:1, :].astype(jnp.int32)
    labs = lab_ref[...]

    rm = jnp.zeros((_TB, _NSITE), jnp.float32)
    tx = jnp.zeros((_TB, _NSITE), jnp.float32)
    ty = jnp.zeros((_TB, _NSITE), jnp.float32)
    tw = jnp.zeros((_TB, _NSITE), jnp.float32)
    th = jnp.zeros((_TB, _NSITE), jnp.float32)
    cbits = jnp.zeros((_TB, _NSITE), jnp.int32)
    for n in range(_NOBJ):
        m = sid[:, n:n + 1] == flat_i
        rm = jnp.where(m, 1.0, rm)
        tx = jnp.where(m, fx[:, n:n + 1], tx)
        ty = jnp.where(m, fy[:, n:n + 1], ty)
        tw = jnp.where(m, twv[:, n:n + 1], tw)
        th = jnp.where(m, thv[:, n:n + 1], th)
        bit = jnp.left_shift(1, labs[:, n:n + 1])
        cbits = jnp.where(m, cbits | bit, cbits)

    px = jax.nn.sigmoid(chan(0))
    py = jax.nn.sigmoid(chan(1))
    pw = jnp.exp(chan(2))
    ph = jnp.exp(chan(3))
    pc = jax.nn.sigmoid(chan(4))

    cell_x = site_ref[1:2, :]
    cell_y = site_ref[2:3, :]
    site_aw = site_ref[3:4, :]
    site_ah = site_ref[4:5, :]
    cpx = cell_x + px
    cpy = cell_y + py
    cpw = site_aw * pw
    cph = site_ah * ph
    px0 = cpx - cpw / 2.0
    py0 = cpy - cph / 2.0
    px1 = cpx + cpw / 2.0
    py1 = cpy + cph / 2.0
    parea = (px1 - px0) * (py1 - py0)
    gc = jnp.zeros((_TB, _NSITE), jnp.float32)
    for n in range(_NOBJ):
        gx0 = cgx0[:, n:n + 1]
        gy0 = cgy0[:, n:n + 1]
        gx1 = cgx1[:, n:n + 1]
        gy1 = cgy1[:, n:n + 1]
        iltx = jnp.maximum(px0, gx0)
        ilty = jnp.maximum(py0, gy0)
        irbx = jnp.minimum(px1, gx1)
        irby = jnp.minimum(py1, gy1)
        iiw = jnp.maximum(irbx - iltx, 0.0)
        iih = jnp.maximum(irby - ilty, 0.0)
        ii = iiw * iih
        ib = (gx1 - gx0) * (gy1 - gy0)
        gc = jnp.maximum(gc, ii / (parea + ib - ii + 1e-10))

    xy_l = jnp.sum(rm * ((tx - px) ** 2 + (ty - py) ** 2))
    wh_l = jnp.sum(rm * ((jnp.sqrt(tw) - jnp.sqrt(pw)) ** 2
                         + (jnp.sqrt(th) - jnp.sqrt(ph)) ** 2))
    d2 = (gc - pc) ** 2
    conf_l = jnp.sum(rm * d2)
    noconf_l = jnp.sum((1.0 - rm) * d2)

    cls_ch = [chan(5 + c) for c in range(_NUM_CLASSES)]
    mx = cls_ch[0]
    for c in range(1, _NUM_CLASSES):
        mx = jnp.maximum(mx, cls_ch[c])
    es = []
    den = jnp.zeros((_TB, _NSITE), jnp.float32)
    for c in range(_NUM_CLASSES):
        e = jnp.exp(cls_ch[c] - mx)
        es.append(e)
        den = den + e
    cls_sq = jnp.zeros((_TB, _NSITE), jnp.float32)
    for c in range(_NUM_CLASSES):
        tcls_c = (jnp.right_shift(cbits, c) & 1).astype(jnp.float32)
        cls_sq = cls_sq + (tcls_c - es[c] / den) ** 2
    rc = jnp.maximum(
        jnp.maximum(rm[:, 0:169], rm[:, 169:338]),
        jnp.maximum(rm[:, 338:507],
                    jnp.maximum(rm[:, 507:676], rm[:, 676:845])))
    rc845 = jnp.concatenate([rc, rc, rc, rc, rc], axis=1)
    cls_l = jnp.sum(rc845 * cls_sq)

    l1 = (5.0 * xy_l).reshape(1, 1)
    l2 = (5.0 * wh_l).reshape(1, 1)
    l3 = conf_l.reshape(1, 1)
    l4 = (0.5 * noconf_l).reshape(1, 1)
    l5 = cls_l.reshape(1, 1)
    tot = l1 + l2 + l3 + l4 + l5
    vec = jnp.concatenate([l1, l2, l3, l4, l5, tot, tot * 0.0, tot * 0.0],
                          axis=1)

    @pl.when(b == 0)
    def _():
        out_ref[...] = vec

    @pl.when(b != 0)
    def _():
        out_ref[...] = out_ref[...] + vec


@jax.jit
def kernel(pred_targets, gt_boxes, gt_labels):
    B = pred_targets.shape[0]
    pred = pred_targets.reshape(B, _GRID * _GRID, _NUM_ANCHORS * 25)
    boxes = gt_boxes.astype(jnp.float32).transpose(2, 0, 1)
    labels = gt_labels.astype(jnp.int32)
    site = jnp.asarray(_site_consts())

    out = pl.pallas_call(
        _loss_kernel,
        grid=(B // _TB,),
        in_specs=[
            pl.BlockSpec((_TB, _GRID * _GRID, _NUM_ANCHORS * 25),
                         lambda b: (b, 0, 0)),
            pl.BlockSpec((4, _TB, _NOBJ), lambda b: (0, b, 0)),
            pl.BlockSpec((_TB, _NOBJ), lambda b: (b, 0)),
            pl.BlockSpec((6, _NSITE), lambda b: (0, 0)),
        ],
        out_specs=pl.BlockSpec((1, 8), lambda b: (0, 0)),
        out_shape=jax.ShapeDtypeStruct((1, 8), jnp.float32),
    )(pred, boxes, labels, site)

    return (out[0, 5], out[0, 0], out[0, 1], out[0, 2], out[0, 3], out[0, 4])


# speedup vs baseline: 1.8043x; 1.0005x over previous
import jax
import jax.numpy as jnp
import numpy as np
from jax.experimental import pallas as pl
from jax.experimental.pallas import tpu as pltpu

_NUM_CLASSES = 20
_GRID = 13
_NUM_ANCHORS = 5
_NCELL = _GRID * _GRID
_NSITE = _NUM_ANCHORS * _NCELL
_NOBJ = 20
_TB = 32
_ANCHORS = np.array(
    [[1.3221, 1.73145], [3.19275, 4.00944], [5.05587, 8.09892],
     [9.47112, 4.84053], [11.2364, 10.0071]], dtype=np.float32)


def _site_consts():
    a = np.arange(_NSITE, dtype=np.int32)
    anc = a // _NCELL
    cell = a % _NCELL
    ci = cell // _GRID
    cj = cell % _GRID
    rows = np.stack([
        a.astype(np.float32),
        ci.astype(np.float32),
        cj.astype(np.float32),
        _ANCHORS[anc, 0],
        _ANCHORS[anc, 1],
        np.zeros(_NSITE, np.float32),
    ], axis=0)
    return rows


def _loss_kernel(pred_ref, box_ref, lab_ref, site_ref, out_ref):
    b = pl.program_id(0)

    pt = jnp.transpose(pred_ref[...], (0, 2, 1))

    def chan(c):
        return jnp.concatenate([pt[:, a * 25 + c, :] for a in range(5)],
                               axis=1)

    x0 = box_ref[0]
    y0 = box_ref[1]
    x1 = box_ref[2]
    y1 = box_ref[3]
    cgx0 = x0 * 13.0
    cgy0 = y0 * 13.0
    cgx1 = x1 * 13.0
    cgy1 = y1 * 13.0
    bcx = ((x0 + x1) / 2.0) * 13.0
    bcy = ((y0 + y1) / 2.0) * 13.0
    bw = (x1 - x0) * 13.0
    bh = (y1 - y0) * 13.0
    cxf = jnp.floor(bcx)
    cyf = jnp.floor(bcy)
    fx = bcx - cxf
    fy = bcy - cyf
    cxi = bcx.astype(jnp.int32)
    cyi = bcy.astype(jnp.int32)

    acx = cxf + 0.5
    acy = cyf + 0.5
    area_b = (cgx1 - cgx0) * (cgy1 - cgy0)
    best_iou = jnp.full_like(bcx, -1.0)
    best_j = jnp.zeros_like(cxi)
    for a in range(_NUM_ANCHORS):
        aw = float(_ANCHORS[a, 0])
        ah = float(_ANCHORS[a, 1])
        ax0 = acx - aw / 2.0
        ay0 = acy - ah / 2.0
        ax1 = acx + aw / 2.0
        ay1 = acy + ah / 2.0
        ltx = jnp.maximum(ax0, cgx0)
        lty = jnp.maximum(ay0, cgy0)
        rbx = jnp.minimum(ax1, cgx1)
        rby = jnp.minimum(ay1, cgy1)
        iw = jnp.maximum(rbx - ltx, 0.0)
        ih = jnp.maximum(rby - lty, 0.0)
        inter = iw * ih
        area_a = (ax1 - ax0) * (ay1 - ay0)
        iou_a = inter / (area_a + area_b - inter + 1e-10)
        better = iou_a > best_iou
        best_j = jnp.where(better, a, best_j)
        best_iou = jnp.where(better, iou_a, best_iou)
    aw_sel = jnp.zeros_like(bcx)
    ah_sel = jnp.zeros_like(bcx)
    for a in range(_NUM_ANCHORS):
        aw_sel = jnp.where(best_j == a, float(_ANCHORS[a, 0]), aw_sel)
        ah_sel = jnp.where(best_j == a, float(_ANCHORS[a, 1]), ah_sel)
    twv = bw / aw_sel
    thv = bh / ah_sel
    sid = best_j * _NCELL + cxi * _GRID + cyi

    flat_i = site_ref[0:1, :].astype(jnp.int32)
    labs = lab_ref[...]

    rm = jnp.zeros((_TB, _NSITE), jnp.float32)
    tx = jnp.zeros((_TB, _NSITE), jnp.float32)
    ty = jnp.zeros((_TB, _NSITE), jnp.float32)
    tw = jnp.zeros((_TB, _NSITE), jnp.float32)
    th = jnp.zeros((_TB, _NSITE), jnp.float32)
    cbits = jnp.zeros((_TB, _NSITE), jnp.int32)
    for n in range(_NOBJ):
        m = sid[:, n:n + 1] == flat_i
        rm = jnp.where(m, 1.0, rm)
        tx = jnp.where(m, fx[:, n:n + 1], tx)
        ty = jnp.where(m, fy[:, n:n + 1], ty)
        tw = jnp.where(m, twv[:, n:n + 1], tw)
        th = jnp.where(m, thv[:, n:n + 1], th)
        bit = jnp.left_shift(1, labs[:, n:n + 1])
        cbits = jnp.where(m, cbits | bit, cbits)

    px = jax.nn.sigmoid(chan(0))
    py = jax.nn.sigmoid(chan(1))
    pw = jnp.exp(chan(2))
    ph = jnp.exp(chan(3))
    pc = jax.nn.sigmoid(chan(4))

    cell_x = site_ref[1:2, :]
    cell_y = site_ref[2:3, :]
    site_aw = site_ref[3:4, :]
    site_ah = site_ref[4:5, :]
    cpx = cell_x + px
    cpy = cell_y + py
    cpw = site_aw * pw
    cph = site_ah * ph
    px0 = cpx - cpw / 2.0
    py0 = cpy - cph / 2.0
    px1 = cpx + cpw / 2.0
    py1 = cpy + cph / 2.0
    parea = (px1 - px0) * (py1 - py0)
    gc = jnp.zeros((_TB, _NSITE), jnp.float32)
    for n in range(_NOBJ):
        gx0 = cgx0[:, n:n + 1]
        gy0 = cgy0[:, n:n + 1]
        gx1 = cgx1[:, n:n + 1]
        gy1 = cgy1[:, n:n + 1]
        iltx = jnp.maximum(px0, gx0)
        ilty = jnp.maximum(py0, gy0)
        irbx = jnp.minimum(px1, gx1)
        irby = jnp.minimum(py1, gy1)
        iiw = jnp.maximum(irbx - iltx, 0.0)
        iih = jnp.maximum(irby - ilty, 0.0)
        ii = iiw * iih
        ib = (gx1 - gx0) * (gy1 - gy0)
        gc = jnp.maximum(gc, ii / (parea + ib - ii + 1e-10))

    xy_l = jnp.sum(rm * ((tx - px) ** 2 + (ty - py) ** 2))
    wh_l = jnp.sum(rm * ((jnp.sqrt(tw) - jnp.sqrt(pw)) ** 2
                         + (jnp.sqrt(th) - jnp.sqrt(ph)) ** 2))
    d2 = (gc - pc) ** 2
    conf_l = jnp.sum(rm * d2)
    noconf_l = jnp.sum((1.0 - rm) * d2)

    cls_ch = [chan(5 + c) for c in range(_NUM_CLASSES)]
    mx = cls_ch[0]
    for c in range(1, _NUM_CLASSES):
        mx = jnp.maximum(mx, cls_ch[c])
    es = []
    den = jnp.zeros((_TB, _NSITE), jnp.float32)
    for c in range(_NUM_CLASSES):
        e = jnp.exp(cls_ch[c] - mx)
        es.append(e)
        den = den + e
    cls_sq = jnp.zeros((_TB, _NSITE), jnp.float32)
    for c in range(_NUM_CLASSES):
        tcls_c = (jnp.right_shift(cbits, c) & 1).astype(jnp.float32)
        cls_sq = cls_sq + (tcls_c - es[c] / den) ** 2
    rc = jnp.maximum(
        jnp.maximum(rm[:, 0:169], rm[:, 169:338]),
        jnp.maximum(rm[:, 338:507],
                    jnp.maximum(rm[:, 507:676], rm[:, 676:845])))
    rc845 = jnp.concatenate([rc, rc, rc, rc, rc], axis=1)
    cls_l = jnp.sum(rc845 * cls_sq)

    l1 = (5.0 * xy_l).reshape(1, 1)
    l2 = (5.0 * wh_l).reshape(1, 1)
    l3 = conf_l.reshape(1, 1)
    l4 = (0.5 * noconf_l).reshape(1, 1)
    l5 = cls_l.reshape(1, 1)
    tot = l1 + l2 + l3 + l4 + l5
    vec = jnp.concatenate([l1, l2, l3, l4, l5, tot, tot * 0.0, tot * 0.0],
                          axis=1)

    @pl.when(b == 0)
    def _():
        out_ref[...] = vec

    @pl.when(b != 0)
    def _():
        out_ref[...] = out_ref[...] + vec


@jax.jit
def kernel(pred_targets, gt_boxes, gt_labels):
    B = pred_targets.shape[0]
    pred = pred_targets.reshape(B, _NCELL, _NUM_ANCHORS * 25)
    boxes = gt_boxes.astype(jnp.float32).transpose(2, 0, 1)
    labels = gt_labels.astype(jnp.int32)
    site = jnp.asarray(_site_consts())

    out = pl.pallas_call(
        _loss_kernel,
        grid=(B // _TB,),
        in_specs=[
            pl.BlockSpec((_TB, _NCELL, _NUM_ANCHORS * 25),
                         lambda b: (b, 0, 0)),
            pl.BlockSpec((4, _TB, _NOBJ), lambda b: (0, b, 0)),
            pl.BlockSpec((_TB, _NOBJ), lambda b: (b, 0)),
            pl.BlockSpec((6, _NSITE), lambda b: (0, 0)),
        ],
        out_specs=pl.BlockSpec((1, 8), lambda b: (0, 0)),
        out_shape=jax.ShapeDtypeStruct((1, 8), jnp.float32),
    )(pred, boxes, labels, site)

    return (out[0, 5], out[0, 0], out[0, 1], out[0, 2], out[0, 3], out[0, 4])
